# baseline (speedup 1.0000x reference)
import functools

import jax
import jax.numpy as jnp
from jax import lax
from jax.experimental import pallas as pl
from jax.experimental.pallas import tpu as pltpu
from jax.experimental.pallas import tpu_sc as plsc

B = 16384
F = 26
E = 16
NC = 2
NS = 16
NW = NC * NS
BPW = B // NW
CHUNK = 64
NCHUNK = BPW // CHUNK
RPC = CHUNK * F


def _sc_body(table, idx_hbm, val_hbm, out_hbm,
             idx_v, val_v, rows_a, rows_b, out_v, sem_a, sem_b):
    c = lax.axis_index("c")
    s = lax.axis_index("s")
    wid = s * NC + c
    base = wid * BPW

    pltpu.sync_copy(idx_hbm.at[pl.ds(base, BPW), :], idx_v)
    pltpu.sync_copy(val_hbm.at[pl.ds(base, BPW), :], val_v)

    bufs = (rows_a, rows_b)
    sems = (sem_a, sem_b)

    def gather_desc(chunk, slot, i):
        return pltpu.make_async_copy(
            table.at[idx_v.at[chunk * CHUNK + i]],
            bufs[slot].at[pl.ds(i * F, F), :],
            sems[slot])

    def fire(chunk, slot):
        def body(i, _):
            gather_desc(chunk, slot, i).start()
            return ()
        lax.fori_loop(0, CHUNK, body, ())

    def drain(chunk, slot):
        def body(i, _):
            gather_desc(chunk, slot, i).wait()
            return ()
        lax.fori_loop(0, CHUNK, body, ())

    lanes = lax.iota(jnp.int32, E)

    def compute(chunk, slot):
        rows = bufs[slot]

        def body(i, _):
            b = chunk * CHUNK + i
            r0 = i * F
            brow = jnp.broadcast_to(b, (E,))
            v_lo = plsc.load_gather(val_v, [brow, lanes])
            v_hi = plsc.load_gather(val_v, [brow, lanes + (F - E)])
            acc0 = jnp.zeros((E,), jnp.float32)
            acc1 = jnp.zeros((E,), jnp.float32)
            sq0 = jnp.zeros((E,), jnp.float32)
            sq1 = jnp.zeros((E,), jnp.float32)
            for f in range(F):
                row = rows[r0 + f, :]
                scalar = v_lo[f] if f < E else v_hi[f - (F - E)]
                wv = row * jnp.broadcast_to(scalar, (E,))
                if f % 2 == 0:
                    acc0 = acc0 + wv
                    sq0 = sq0 + wv * wv
                else:
                    acc1 = acc1 + wv
                    sq1 = sq1 + wv * wv
            acc = acc0 + acc1
            sq = sq0 + sq1
            out_v[b, :] = (acc * acc - sq) * 0.5
            return ()

        lax.fori_loop(0, CHUNK, body, ())

    fire(0, 0)
    for chunk in range(NCHUNK):
        slot = chunk % 2
        drain(chunk, slot)
        if chunk + 1 < NCHUNK:
            fire(chunk + 1, 1 - slot)
        compute(chunk, slot)

    pltpu.sync_copy(out_v, out_hbm.at[pl.ds(base, BPW), :])


@jax.jit
def _bi_interaction_sc(feat_index, feat_value, emb_table):
    mesh = plsc.VectorSubcoreMesh(core_axis_name="c", subcore_axis_name="s")
    fn = pl.kernel(
        _sc_body,
        out_type=jax.ShapeDtypeStruct((B, E), jnp.float32),
        mesh=mesh,
        compiler_params=pltpu.CompilerParams(
            use_tc_tiling_on_sc=False, needs_layout_passes=False),
        scratch_types=[
            pltpu.VMEM((BPW, F), jnp.int32),
            pltpu.VMEM((BPW, F), jnp.float32),
            pltpu.VMEM((RPC, E), jnp.float32),
            pltpu.VMEM((RPC, E), jnp.float32),
            pltpu.VMEM((BPW, E), jnp.float32),
            pltpu.SemaphoreType.DMA,
            pltpu.SemaphoreType.DMA,
        ],
    )
    return fn(emb_table, feat_index.astype(jnp.int32), feat_value)


def _mlp_body(bi_ref, w1_ref, b1_ref, w2_ref, b2_ref, wo_ref, bo_ref, out_ref):
    x = bi_ref[...]
    h = jnp.dot(x, w1_ref[...], preferred_element_type=jnp.float32)
    h = jnp.maximum(h + b1_ref[...], 0.0)
    h = jnp.dot(h, w2_ref[...], preferred_element_type=jnp.float32)
    h = jnp.maximum(h + b2_ref[...], 0.0)
    o = jnp.sum(h * wo_ref[...], axis=1, keepdims=True) + bo_ref[...]
    out_ref[...] = 1.0 / (1.0 + jnp.exp(-o))


@jax.jit
def _mlp_tc(bi, W1, b1, W2, b2, Wo, bo):
    nblk = 8
    blk = B // nblk
    return pl.pallas_call(
        _mlp_body,
        grid=(nblk,),
        in_specs=[
            pl.BlockSpec((blk, E), lambda i: (i, 0)),
            pl.BlockSpec((E, 32), lambda i: (0, 0)),
            pl.BlockSpec((1, 32), lambda i: (0, 0)),
            pl.BlockSpec((32, 32), lambda i: (0, 0)),
            pl.BlockSpec((1, 32), lambda i: (0, 0)),
            pl.BlockSpec((1, 32), lambda i: (0, 0)),
            pl.BlockSpec((1, 1), lambda i: (0, 0)),
        ],
        out_specs=pl.BlockSpec((blk, 1), lambda i: (i, 0)),
        out_shape=jax.ShapeDtypeStruct((B, 1), jnp.float32),
    )(bi, W1, b1.reshape(1, 32), W2, b2.reshape(1, 32),
      Wo.reshape(1, 32), bo.reshape(1, 1))


def kernel(feat_index, feat_value, emb_table, W1, b1, W2, b2, Wo, bo):
    bi = _bi_interaction_sc(feat_index, feat_value, emb_table)
    return _mlp_tc(bi, W1, b1, W2, b2, Wo, bo)


# speedup vs baseline: 1.2541x; 1.2541x over previous
import functools

import jax
import jax.numpy as jnp
from jax import lax
from jax.experimental import pallas as pl
from jax.experimental.pallas import tpu as pltpu
from jax.experimental.pallas import tpu_sc as plsc

B = 16384
F = 26
E = 16
NC = 2
NS = 16
NW = NC * NS
BPW = B // NW
CHUNK = 64
NCHUNK = BPW // CHUNK
RPC = CHUNK * F


def _sc_body(table, idx_hbm, val_hbm, out_hbm,
             idx_v, val_v, rows_a, rows_b, out_v, sem_a, sem_b):
    c = lax.axis_index("c")
    s = lax.axis_index("s")
    wid = s * NC + c
    base = wid * BPW

    pltpu.sync_copy(idx_hbm.at[pl.ds(base, BPW), :], idx_v)
    pltpu.sync_copy(val_hbm.at[pl.ds(base, BPW), :], val_v)

    bufs = (rows_a, rows_b)
    sems = (sem_a, sem_b)

    def gather_desc(chunk, slot, i):
        return pltpu.make_async_copy(
            table.at[idx_v.at[chunk * CHUNK + i]],
            bufs[slot].at[pl.ds(i * F, F), :],
            sems[slot])

    def fire(chunk, slot):
        def body(i, _):
            gather_desc(chunk, slot, i).start()
            return ()
        lax.fori_loop(0, CHUNK, body, ())

    def drain(chunk, slot):
        def body(i, _):
            gather_desc(chunk, slot, i).wait()
            return ()
        lax.fori_loop(0, CHUNK, body, ())

    lanes = lax.iota(jnp.int32, E)

    def compute(chunk, slot):
        rows = bufs[slot]

        def body(i, _):
            b = chunk * CHUNK + i
            r0 = i * F
            brow = jnp.broadcast_to(b, (E,))
            v_lo = plsc.load_gather(val_v, [brow, lanes])
            v_hi = plsc.load_gather(val_v, [brow, lanes + (F - E)])
            acc0 = jnp.zeros((E,), jnp.float32)
            acc1 = jnp.zeros((E,), jnp.float32)
            sq0 = jnp.zeros((E,), jnp.float32)
            sq1 = jnp.zeros((E,), jnp.float32)
            for f in range(F):
                row = rows[r0 + f, :]
                scalar = v_lo[f] if f < E else v_hi[f - (F - E)]
                wv = row * jnp.broadcast_to(scalar, (E,))
                if f % 2 == 0:
                    acc0 = acc0 + wv
                    sq0 = sq0 + wv * wv
                else:
                    acc1 = acc1 + wv
                    sq1 = sq1 + wv * wv
            acc = acc0 + acc1
            sq = sq0 + sq1
            out_v[b, :] = (acc * acc - sq) * 0.5
            return ()

        lax.fori_loop(0, CHUNK, body, ())

    fire(0, 0)
    for chunk in range(NCHUNK):
        slot = chunk % 2
        drain(chunk, slot)
        if chunk + 1 < NCHUNK:
            fire(chunk + 1, 1 - slot)
        compute(chunk, slot)

    pltpu.sync_copy(out_v, out_hbm.at[pl.ds(base, BPW), :])


def _relayout_body(in_ref, out_ref):
    x = in_ref[...]
    ch = x.shape[1]
    y = jnp.transpose(x)
    y4 = y.reshape(ch // 64, 8, 8, E)
    cols = [y4[:, :, s, :].reshape(ch // 8, E) for s in range(8)]
    out_ref[...] = jnp.concatenate(cols, axis=1)


@jax.jit
def _linearize_table_tc(emb_table):
    tt = jnp.transpose(emb_table)
    grid = 123
    ch = 8192
    lin = pl.pallas_call(
        _relayout_body,
        grid=(grid,),
        in_specs=[pl.BlockSpec((E, ch), lambda i: (0, i))],
        out_specs=pl.BlockSpec((ch // 8, 128), lambda i: (i, 0)),
        out_shape=jax.ShapeDtypeStruct((grid * ch // 8, 128), jnp.float32),
    )(tt)
    return lin.reshape(grid * ch, E)


@jax.jit
def _bi_interaction_sc(feat_index, feat_value, emb_table):
    mesh = plsc.VectorSubcoreMesh(core_axis_name="c", subcore_axis_name="s")
    fn = pl.kernel(
        _sc_body,
        out_type=jax.ShapeDtypeStruct((B, E), jnp.float32),
        mesh=mesh,
        compiler_params=pltpu.CompilerParams(
            use_tc_tiling_on_sc=False, needs_layout_passes=False),
        scratch_types=[
            pltpu.VMEM((BPW, F), jnp.int32),
            pltpu.VMEM((BPW, F), jnp.float32),
            pltpu.VMEM((RPC, E), jnp.float32),
            pltpu.VMEM((RPC, E), jnp.float32),
            pltpu.VMEM((BPW, E), jnp.float32),
            pltpu.SemaphoreType.DMA,
            pltpu.SemaphoreType.DMA,
        ],
    )
    return fn(emb_table, feat_index.astype(jnp.int32), feat_value)


def _mlp_body(bi_ref, w1_ref, b1_ref, w2_ref, b2_ref, wo_ref, bo_ref, out_ref):
    x = bi_ref[...]
    h = jnp.dot(x, w1_ref[...], preferred_element_type=jnp.float32)
    h = jnp.maximum(h + b1_ref[...], 0.0)
    h = jnp.dot(h, w2_ref[...], preferred_element_type=jnp.float32)
    h = jnp.maximum(h + b2_ref[...], 0.0)
    o = jnp.sum(h * wo_ref[...], axis=1, keepdims=True) + bo_ref[...]
    out_ref[...] = 1.0 / (1.0 + jnp.exp(-o))


@jax.jit
def _mlp_tc(bi, W1, b1, W2, b2, Wo, bo):
    nblk = 8
    blk = B // nblk
    return pl.pallas_call(
        _mlp_body,
        grid=(nblk,),
        in_specs=[
            pl.BlockSpec((blk, E), lambda i: (i, 0)),
            pl.BlockSpec((E, 32), lambda i: (0, 0)),
            pl.BlockSpec((1, 32), lambda i: (0, 0)),
            pl.BlockSpec((32, 32), lambda i: (0, 0)),
            pl.BlockSpec((1, 32), lambda i: (0, 0)),
            pl.BlockSpec((1, 32), lambda i: (0, 0)),
            pl.BlockSpec((1, 1), lambda i: (0, 0)),
        ],
        out_specs=pl.BlockSpec((blk, 1), lambda i: (i, 0)),
        out_shape=jax.ShapeDtypeStruct((B, 1), jnp.float32),
    )(bi, W1, b1.reshape(1, 32), W2, b2.reshape(1, 32),
      Wo.reshape(1, 32), bo.reshape(1, 1))


@jax.jit
def _nfm(feat_index, feat_value, emb_table, W1, b1, W2, b2, Wo, bo):
    table_lin = _linearize_table_tc(emb_table)
    bi = _bi_interaction_sc(feat_index, feat_value, table_lin)
    return _mlp_tc(bi, W1, b1, W2, b2, Wo, bo)


def kernel(feat_index, feat_value, emb_table, W1, b1, W2, b2, Wo, bo):
    return _nfm(feat_index, feat_value, emb_table, W1, b1, W2, b2, Wo, bo)
